# baseline (speedup 1.0000x reference)
import functools

import jax
import jax.numpy as jnp
from jax import lax
from jax.experimental import pallas as pl
from jax.experimental.pallas import tpu as pltpu
from jax.experimental.pallas import tpu_sc as plsc

B = 4096
S = 200
H = 32
NL = 2
NC = 2
NS = 16
NW = NC * NS
BPW = B // NW
GL = BPW
CH_J = 10
CH_ROWS = CH_J * GL
NCH = S // CH_J
NBUF = 3

TC = 32768
TC8 = TC // 8
TNB = 31
VP = TNB * TC


def _sc_pool(ids_t, table):
    mesh = plsc.VectorSubcoreMesh(
        core_axis_name="c", subcore_axis_name="s",
        num_cores=NC, num_subcores=NS)

    @functools.partial(
        pl.kernel, mesh=mesh,
        compiler_params=pltpu.CompilerParams(
            use_tc_tiling_on_sc=False, needs_layout_passes=False),
        out_type=jax.ShapeDtypeStruct((B, H), jnp.float32),
        scratch_types=[
            pltpu.VMEM((S, BPW), jnp.int32),
            pltpu.VMEM((CH_ROWS, 16), jnp.int32),
            pltpu.VMEM((CH_ROWS, 16), jnp.int32),
            pltpu.VMEM((CH_ROWS, 16), jnp.int32),
            pltpu.VMEM((BPW, H), jnp.float32),
            pltpu.SemaphoreType.DMA,
            pltpu.SemaphoreType.DMA,
            pltpu.SemaphoreType.DMA,
        ],
    )
    def pool(ids_hbm, table_hbm, out_hbm, tbuf, rows0, rows1, rows2,
             acc_v, sem0, sem1, sem2):
        w = lax.axis_index("s") * NC + lax.axis_index("c")
        rows = (rows0, rows1, rows2)
        sems = (sem0, sem1, sem2)

        pltpu.sync_copy(ids_hbm.at[:, pl.ds(w * BPW, BPW)], tbuf)

        def prep_chunk(ch):
            def pbody(j, c):
                for k in range(BPW // 16):
                    x = tbuf[j, pl.ds(16 * k, 16)]
                    tbuf[j, pl.ds(16 * k, 16)] = (
                        ((x >> 15) << 15) | ((x & 0xFFF) << 3)
                        | ((x >> 12) & 7))
                return c
            lax.fori_loop(ch * CH_J, (ch + 1) * CH_J, pbody, 0)

        def fire(ch, bi):
            for g in range(CH_J):
                pltpu.async_copy(
                    table_hbm.at[tbuf.at[ch * CH_J + g]],
                    rows[bi].at[pl.ds(g * GL, GL)],
                    sems[bi])

        def drain(bi):
            pltpu.make_async_copy(
                table_hbm.at[pl.ds(0, CH_ROWS)], rows[bi], sems[bi]).wait()

        hi_mask = jnp.full((16,), -65536, jnp.int32)

        def consume(ch, bi):
            def ebody(e, c, _bi=bi, _init=(ch == 0)):
                if _init:
                    a_lo = jnp.zeros((16,), jnp.float32)
                    a_hi = jnp.zeros((16,), jnp.float32)
                else:
                    a_lo = acc_v[e, 0:16]
                    a_hi = acc_v[e, 16:32]
                for j in range(CH_J):
                    x = rows[_bi][j * GL + e, 0:16]
                    a_lo = a_lo + plsc.bitcast(x << 16, jnp.float32)
                    a_hi = a_hi + plsc.bitcast(x & hi_mask, jnp.float32)
                acc_v[e, 0:16] = a_lo
                acc_v[e, 16:32] = a_hi
                return c
            lax.fori_loop(0, BPW, ebody, 0)

        for ch in range(NBUF):
            prep_chunk(ch)
            fire(ch, ch)

        for ch in range(NCH):
            bi = ch % NBUF
            drain(bi)
            consume(ch, bi)
            if ch + NBUF < NCH:
                prep_chunk(ch + NBUF)
                fire(ch + NBUF, bi)

        pltpu.sync_copy(acc_v, out_hbm.at[pl.ds(w * BPW, BPW)])

    return pool(ids_t, table)


def _tc_transpose(table_t):
    V = table_t.shape[1]

    def body(x_ref, mask_ref, eye_ref, o_ref):
        xb = x_ref[...].astype(jnp.bfloat16) * mask_ref[0]
        xs_lo = jnp.concatenate(
            [xb[0:16, m * TC8:(m + 1) * TC8] for m in range(8)], axis=0)
        xs_hi = jnp.concatenate(
            [xb[16:32, m * TC8:(m + 1) * TC8] for m in range(8)], axis=0)
        eye = eye_ref[...]
        tl = jax.lax.dot_general(
            xs_lo, eye, (((0,), (0,)), ((), ())),
            preferred_element_type=jnp.float32)
        th = jax.lax.dot_general(
            xs_hi, eye, (((0,), (0,)), ((), ())),
            preferred_element_type=jnp.float32)
        lo = jax.lax.bitcast_convert_type(tl, jnp.int32)
        hi = jax.lax.bitcast_convert_type(th, jnp.int32)
        o_ref[...] = jax.lax.shift_right_logical(lo, 16) | hi

    eye = jnp.eye(4 * H, dtype=jnp.bfloat16)
    mask = (jnp.arange(TNB * TC, dtype=jnp.int32) < V).astype(
        jnp.bfloat16).reshape(TNB, 1, TC)
    return pl.pallas_call(
        body,
        grid=(TNB,),
        in_specs=[pl.BlockSpec((H, TC), lambda i: (0, i)),
                  pl.BlockSpec((1, 1, TC), lambda i: (i, 0, 0)),
                  pl.BlockSpec((4 * H, 4 * H), lambda i: (0, 0))],
        out_specs=pl.BlockSpec((TC8, 4 * H), lambda i: (i, 0)),
        out_shape=jax.ShapeDtypeStruct((VP // 8, 4 * H), jnp.int32),
    )(table_t, mask, eye)


def _tc_head(x, w2, b2):
    def body(x_ref, w_ref, b_ref, o_ref):
        o_ref[...] = jnp.dot(
            x_ref[...], w_ref[...],
            preferred_element_type=jnp.float32) + b_ref[...]

    return pl.pallas_call(
        body,
        out_shape=jax.ShapeDtypeStruct((B, NL), jnp.float32),
    )(x, w2, b2)


def kernel(input_ids, embed_table, W, b):
    packed = _tc_transpose(embed_table.T)
    table_rm = packed.reshape(VP, 16)
    sums = _sc_pool(input_ids.T, table_rm)
    w2 = W * jnp.float32(1.0 / S)
    return _tc_head(sums, w2, b.reshape(1, NL))


# speedup vs baseline: 1.0457x; 1.0457x over previous
import functools

import jax
import jax.numpy as jnp
from jax import lax
from jax.experimental import pallas as pl
from jax.experimental.pallas import tpu as pltpu
from jax.experimental.pallas import tpu_sc as plsc

B = 4096
S = 200
H = 32
NL = 2
NC = 2
NS = 16
NW = NC * NS
BPW = B // NW
GL = BPW
CH_J = 20
CH_ROWS = CH_J * GL
NCH = S // CH_J
NBUF = 2

TC = 32768
TC8 = TC // 8
TNB = 31
VP = TNB * TC


def _sc_pool(ids_t, table):
    mesh = plsc.VectorSubcoreMesh(
        core_axis_name="c", subcore_axis_name="s",
        num_cores=NC, num_subcores=NS)

    @functools.partial(
        pl.kernel, mesh=mesh,
        compiler_params=pltpu.CompilerParams(
            use_tc_tiling_on_sc=False, needs_layout_passes=False),
        out_type=jax.ShapeDtypeStruct((B, H), jnp.float32),
        scratch_types=[
            pltpu.VMEM((S, BPW), jnp.int32),
            pltpu.VMEM((CH_ROWS, 16), jnp.int32),
            pltpu.VMEM((CH_ROWS, 16), jnp.int32),
            pltpu.VMEM((BPW, H), jnp.float32),
            pltpu.SemaphoreType.DMA,
            pltpu.SemaphoreType.DMA,
        ],
    )
    def pool(ids_hbm, table_hbm, out_hbm, tbuf, rows0, rows1,
             acc_v, sem0, sem1):
        w = lax.axis_index("s") * NC + lax.axis_index("c")
        rows = (rows0, rows1)
        sems = (sem0, sem1)

        pltpu.sync_copy(ids_hbm.at[:, pl.ds(w * BPW, BPW)], tbuf)

        def prep_chunk(ch):
            def pbody(j, c):
                for k in range(BPW // 16):
                    x = tbuf[j, pl.ds(16 * k, 16)]
                    tbuf[j, pl.ds(16 * k, 16)] = (
                        ((x >> 15) << 15) | ((x & 0xFFF) << 3)
                        | ((x >> 12) & 7))
                return c
            lax.fori_loop(ch * CH_J, (ch + 1) * CH_J, pbody, 0)

        def fire(ch, bi):
            for g in range(CH_J):
                pltpu.async_copy(
                    table_hbm.at[tbuf.at[ch * CH_J + g]],
                    rows[bi].at[pl.ds(g * GL, GL)],
                    sems[bi])

        def drain(bi):
            pltpu.make_async_copy(
                table_hbm.at[pl.ds(0, CH_ROWS)], rows[bi], sems[bi]).wait()

        hi_mask = jnp.full((16,), -65536, jnp.int32)

        def consume(ch, bi):
            def ebody(e, c, _bi=bi, _init=(ch == 0)):
                if _init:
                    a_lo = jnp.zeros((16,), jnp.float32)
                    a_hi = jnp.zeros((16,), jnp.float32)
                else:
                    a_lo = acc_v[e, 0:16]
                    a_hi = acc_v[e, 16:32]
                for j in range(CH_J):
                    x = rows[_bi][j * GL + e, 0:16]
                    a_lo = a_lo + plsc.bitcast(x << 16, jnp.float32)
                    a_hi = a_hi + plsc.bitcast(x & hi_mask, jnp.float32)
                acc_v[e, 0:16] = a_lo
                acc_v[e, 16:32] = a_hi
                return c
            lax.fori_loop(0, BPW, ebody, 0)

        for ch in range(NBUF):
            prep_chunk(ch)
            fire(ch, ch)

        for ch in range(NCH):
            bi = ch % NBUF
            drain(bi)
            consume(ch, bi)
            if ch + NBUF < NCH:
                prep_chunk(ch + NBUF)
                fire(ch + NBUF, bi)

        pltpu.sync_copy(acc_v, out_hbm.at[pl.ds(w * BPW, BPW)])

    return pool(ids_t, table)


def _tc_transpose(table_t):
    V = table_t.shape[1]

    def body(x_ref, mask_ref, eye_ref, o_ref):
        xb = x_ref[...].astype(jnp.bfloat16) * mask_ref[0]
        xs_lo = jnp.concatenate(
            [xb[0:16, m * TC8:(m + 1) * TC8] for m in range(8)], axis=0)
        xs_hi = jnp.concatenate(
            [xb[16:32, m * TC8:(m + 1) * TC8] for m in range(8)], axis=0)
        eye = eye_ref[...]
        tl = jax.lax.dot_general(
            xs_lo, eye, (((0,), (0,)), ((), ())),
            preferred_element_type=jnp.float32)
        th = jax.lax.dot_general(
            xs_hi, eye, (((0,), (0,)), ((), ())),
            preferred_element_type=jnp.float32)
        lo = jax.lax.bitcast_convert_type(tl, jnp.int32)
        hi = jax.lax.bitcast_convert_type(th, jnp.int32)
        o_ref[...] = jax.lax.shift_right_logical(lo, 16) | hi

    eye = jnp.eye(4 * H, dtype=jnp.bfloat16)
    mask = (jnp.arange(TNB * TC, dtype=jnp.int32) < V).astype(
        jnp.bfloat16).reshape(TNB, 1, TC)
    return pl.pallas_call(
        body,
        grid=(TNB,),
        in_specs=[pl.BlockSpec((H, TC), lambda i: (0, i)),
                  pl.BlockSpec((1, 1, TC), lambda i: (i, 0, 0)),
                  pl.BlockSpec((4 * H, 4 * H), lambda i: (0, 0))],
        out_specs=pl.BlockSpec((TC8, 4 * H), lambda i: (i, 0)),
        out_shape=jax.ShapeDtypeStruct((VP // 8, 4 * H), jnp.int32),
    )(table_t, mask, eye)


def _tc_head(x, w2, b2):
    def body(x_ref, w_ref, b_ref, o_ref):
        o_ref[...] = jnp.dot(
            x_ref[...], w_ref[...],
            preferred_element_type=jnp.float32) + b_ref[...]

    return pl.pallas_call(
        body,
        out_shape=jax.ShapeDtypeStruct((B, NL), jnp.float32),
    )(x, w2, b2)


def kernel(input_ids, embed_table, W, b):
    packed = _tc_transpose(embed_table.T)
    table_rm = packed.reshape(VP, 16)
    sums = _sc_pool(input_ids.T, table_rm)
    w2 = W * jnp.float32(1.0 / S)
    return _tc_head(sums, w2, b.reshape(1, NL))
